# Initial kernel scaffold; baseline (speedup 1.0000x reference)
import jax
import jax.numpy as jnp
from jax.experimental import pallas as pl


def kernel(x):
    raise NotImplementedError("write your pallas kernel here")


# speedup vs baseline: 1.7815x; 1.7815x over previous
import jax
import jax.numpy as jnp
from jax import lax
from jax.experimental import pallas as pl
from jax.experimental.pallas import tpu as pltpu


def _cdiv(a, b):
    return -(-a // b)


def _subsample_kernel(x_ref, o_ref):
    pt, ho, wo = o_ref.shape
    w = x_ref.shape[-1]
    rows = lax.broadcasted_iota(jnp.int32, (w, wo), 0)
    cols = lax.broadcasted_iota(jnp.int32, (w, wo), 1)
    sel = (rows == 2 * cols).astype(x_ref.dtype)
    xv = x_ref[:, :, 0, 0, :].reshape(pt * ho, w)
    out = jnp.dot(xv, sel, preferred_element_type=jnp.float32)
    o_ref[...] = out.reshape(pt, ho, wo).astype(o_ref.dtype)


def kernel(x):
    n, c, h, w = x.shape
    ho = (h - 1) // 2 + 1
    wo = (w - 1) // 2 + 1
    p = n * c

    x5 = x.reshape(p, ho, 2, 1, w)

    pt = 128
    grid = (_cdiv(p, pt),)

    out = pl.pallas_call(
        _subsample_kernel,
        out_shape=jax.ShapeDtypeStruct((p, ho, wo), x.dtype),
        grid=grid,
        in_specs=[pl.BlockSpec((pt, ho, 1, 1, w), lambda i: (i, 0, 0, 0, 0))],
        out_specs=pl.BlockSpec((pt, ho, wo), lambda i: (i, 0, 0)),
        compiler_params=pltpu.CompilerParams(
            dimension_semantics=("parallel",)),
        cost_estimate=pl.CostEstimate(
            flops=2 * p * ho * w * wo, transcendentals=0,
            bytes_accessed=(p * ho * w + p * ho * wo) * x.dtype.itemsize),
    )(x5)
    return [out.reshape(n, c, ho, wo)]
